# Initial kernel scaffold; baseline (speedup 1.0000x reference)
import functools

import jax
import jax.numpy as jnp
from jax import lax
from jax.experimental import pallas as pl
from jax.experimental.pallas import tpu as pltpu
from jax.experimental.pallas import tpu_sc as plsc

YEAR_DIM = 128
MONTH_PAD = 16
D_MODEL = 32
HALF = D_MODEL // 2
N_ROWS = 16384 * 200

NW = 32
CHUNK = 2048
IDX_MINOR = 128
IDX_ROWS_PER_CHUNK = CHUNK // IDX_MINOR
ROWS_PER_WORKER = N_ROWS // NW
CHUNKS_PER_WORKER = ROWS_PER_WORKER // CHUNK
IDX_ROWS_PER_WORKER = ROWS_PER_WORKER // IDX_MINOR


def _table_body(yt_ref, mt_ref, w_ref, b_ref, out_ref):
    yt = yt_ref[...]
    mt = mt_ref[...]
    w = w_ref[...]
    b = b_ref[...]
    dn = (((1,), (1,)), ((), ()))
    yp = lax.dot_general(yt, w[:, :HALF], dn,
                         preferred_element_type=jnp.float32,
                         precision=lax.Precision.HIGHEST)
    mp = lax.dot_general(mt, w[:, HALF:], dn,
                         preferred_element_type=jnp.float32,
                         precision=lax.Precision.HIGHEST) + b
    comb = yp[:, None, :] + mp[None, :, :]
    out_ref[...] = comb.reshape(YEAR_DIM * MONTH_PAD, D_MODEL)


def _build_table(year_table, month_table_padded, w, b2d):
    return pl.pallas_call(
        _table_body,
        out_shape=jax.ShapeDtypeStruct((YEAR_DIM * MONTH_PAD, D_MODEL),
                                       jnp.float32),
    )(year_table, month_table_padded, w, b2d)


_SC_MESH = plsc.VectorSubcoreMesh(core_axis_name="c", subcore_axis_name="s")


@functools.partial(
    pl.kernel,
    out_type=jax.ShapeDtypeStruct((N_ROWS, D_MODEL), jnp.float32),
    mesh=_SC_MESH,
    scratch_types=[
        pltpu.VMEM((IDX_ROWS_PER_CHUNK, IDX_MINOR), jnp.int32),
        pltpu.VMEM((IDX_ROWS_PER_CHUNK, IDX_MINOR), jnp.int32),
        pltpu.VMEM((IDX_ROWS_PER_CHUNK, IDX_MINOR), jnp.int32),
        pltpu.VMEM((CHUNK, D_MODEL), jnp.float32),
        pltpu.SemaphoreType.DMA,
    ],
)
def _sc_lookup(table_hbm, yi_hbm, mi_hbm, out_hbm, yi_v, mi_v, ci_v, rows_v,
               sem):
    wid = lax.axis_index("s") * 2 + lax.axis_index("c")
    idx_row0 = wid * IDX_ROWS_PER_WORKER
    out_row0 = wid * ROWS_PER_WORKER

    def chunk_body(it, carry):
        ib = idx_row0 + it * IDX_ROWS_PER_CHUNK
        pltpu.sync_copy(yi_hbm.at[pl.ds(ib, IDX_ROWS_PER_CHUNK)], yi_v)
        pltpu.sync_copy(mi_hbm.at[pl.ds(ib, IDX_ROWS_PER_CHUNK)], mi_v)
        for j in range(IDX_ROWS_PER_CHUNK):
            for k in range(IDX_MINOR // 16):
                s = pl.ds(k * 16, 16)
                ci_v[j, s] = yi_v[j, s] * MONTH_PAD + mi_v[j, s]
        copies = [
            pltpu.async_copy(
                table_hbm.at[ci_v.at[j]],
                rows_v.at[pl.ds(j * IDX_MINOR, IDX_MINOR)],
                sem,
            )
            for j in range(IDX_ROWS_PER_CHUNK)
        ]
        for c in copies:
            c.wait()
        pltpu.sync_copy(
            rows_v, out_hbm.at[pl.ds(out_row0 + it * CHUNK, CHUNK)])
        return carry

    lax.fori_loop(0, CHUNKS_PER_WORKER, chunk_body, 0)


def kernel(year_indices, month_indices, year_table, month_table, W, b):
    mt_pad = jnp.zeros((MONTH_PAD, HALF), jnp.float32).at[:12].set(month_table)
    table = _build_table(year_table, mt_pad, W, b.reshape(1, D_MODEL))
    yi = year_indices.reshape(N_ROWS // IDX_MINOR, IDX_MINOR).astype(jnp.int32)
    mi = month_indices.reshape(N_ROWS // IDX_MINOR, IDX_MINOR).astype(jnp.int32)
    return _sc_lookup(table, yi, mi)


# speedup vs baseline: 11.1503x; 11.1503x over previous
import functools

import jax
import jax.numpy as jnp
from jax import lax
from jax.experimental import pallas as pl
from jax.experimental.pallas import tpu as pltpu
from jax.experimental.pallas import tpu_sc as plsc

YEAR_DIM = 128
MONTH_PAD = 16
D_MODEL = 32
HALF = D_MODEL // 2
N_ROWS = 16384 * 200

NW = 32
CHUNK = 2048
IDX_MINOR = 128
IDX_ROWS_PER_CHUNK = CHUNK // IDX_MINOR
ROWS_PER_WORKER = N_ROWS // NW
CHUNKS_PER_WORKER = ROWS_PER_WORKER // CHUNK
IDX_ROWS_PER_WORKER = ROWS_PER_WORKER // IDX_MINOR


def _table_body(yt_ref, mt_ref, w_ref, b_ref, out_ref):
    yt = yt_ref[...]
    mt = mt_ref[...]
    w = w_ref[...]
    b = b_ref[...]
    dn = (((1,), (1,)), ((), ()))
    yp = lax.dot_general(yt, w[:, :HALF], dn,
                         preferred_element_type=jnp.float32,
                         precision=lax.Precision.HIGHEST)
    mp = lax.dot_general(mt, w[:, HALF:], dn,
                         preferred_element_type=jnp.float32,
                         precision=lax.Precision.HIGHEST) + b
    comb = yp[:, None, :] + mp[None, :, :]
    out_ref[...] = comb.reshape(YEAR_DIM * MONTH_PAD, D_MODEL)


def _build_table(year_table, month_table_padded, w, b2d):
    return pl.pallas_call(
        _table_body,
        out_shape=jax.ShapeDtypeStruct((YEAR_DIM * MONTH_PAD, D_MODEL),
                                       jnp.float32),
    )(year_table, month_table_padded, w, b2d)


_SC_MESH = plsc.VectorSubcoreMesh(core_axis_name="c", subcore_axis_name="s")


@functools.partial(
    pl.kernel,
    out_type=jax.ShapeDtypeStruct((N_ROWS, D_MODEL), jnp.float32),
    mesh=_SC_MESH,
    compiler_params=pltpu.CompilerParams(use_tc_tiling_on_sc=False),
    scratch_types=[
        pltpu.VMEM((IDX_ROWS_PER_CHUNK, IDX_MINOR), jnp.int32),
        pltpu.VMEM((IDX_ROWS_PER_CHUNK, IDX_MINOR), jnp.int32),
        pltpu.VMEM((IDX_ROWS_PER_CHUNK, IDX_MINOR), jnp.int32),
        pltpu.VMEM((CHUNK, D_MODEL), jnp.float32),
        pltpu.SemaphoreType.DMA,
    ],
)
def _sc_lookup(table_hbm, yi_hbm, mi_hbm, out_hbm, yi_v, mi_v, ci_v, rows_v,
               sem):
    wid = lax.axis_index("s") * 2 + lax.axis_index("c")
    idx_row0 = wid * IDX_ROWS_PER_WORKER
    out_row0 = wid * ROWS_PER_WORKER

    def chunk_body(it, carry):
        ib = idx_row0 + it * IDX_ROWS_PER_CHUNK
        pltpu.sync_copy(yi_hbm.at[pl.ds(ib, IDX_ROWS_PER_CHUNK)], yi_v)
        pltpu.sync_copy(mi_hbm.at[pl.ds(ib, IDX_ROWS_PER_CHUNK)], mi_v)
        for j in range(IDX_ROWS_PER_CHUNK):
            for k in range(IDX_MINOR // 16):
                s = pl.ds(k * 16, 16)
                ci_v[j, s] = yi_v[j, s] * MONTH_PAD + mi_v[j, s]
        copies = [
            pltpu.async_copy(
                table_hbm.at[ci_v.at[j]],
                rows_v.at[pl.ds(j * IDX_MINOR, IDX_MINOR)],
                sem,
            )
            for j in range(IDX_ROWS_PER_CHUNK)
        ]
        for c in copies:
            c.wait()
        pltpu.sync_copy(
            rows_v, out_hbm.at[pl.ds(out_row0 + it * CHUNK, CHUNK)])
        return carry

    lax.fori_loop(0, CHUNKS_PER_WORKER, chunk_body, 0)


def kernel(year_indices, month_indices, year_table, month_table, W, b):
    mt_pad = jnp.zeros((MONTH_PAD, HALF), jnp.float32).at[:12].set(month_table)
    table = _build_table(year_table, mt_pad, W, b.reshape(1, D_MODEL))
    yi = year_indices.reshape(N_ROWS // IDX_MINOR, IDX_MINOR).astype(jnp.int32)
    mi = month_indices.reshape(N_ROWS // IDX_MINOR, IDX_MINOR).astype(jnp.int32)
    return _sc_lookup(table, yi, mi)


# speedup vs baseline: 11.3794x; 1.0206x over previous
import functools

import jax
import jax.numpy as jnp
from jax import lax
from jax.experimental import pallas as pl
from jax.experimental.pallas import tpu as pltpu
from jax.experimental.pallas import tpu_sc as plsc

YEAR_DIM = 128
MONTH_PAD = 16
D_MODEL = 32
HALF = D_MODEL // 2
B_ROWS = 16384
L_SEQ = 200
N_ROWS = B_ROWS * L_SEQ

NW = 32
KB_TOTAL = B_ROWS // 128
KB_PER_W = KB_TOTAL // NW
TILE_ROWS = 128 * L_SEQ
SUB_IDX = 5
SUB_ROWS = SUB_IDX * 128
SUBS_PER_TILE = L_SEQ // SUB_IDX
PAIRS_PER_TILE = SUBS_PER_TILE // 2


def _table_body(yt_ref, mt_ref, w_ref, b_ref, out_ref):
    yt = yt_ref[...]
    mt = mt_ref[...]
    w = w_ref[...]
    b = b_ref[...]
    dn = (((1,), (1,)), ((), ()))
    yp = lax.dot_general(yt, w[:, :HALF], dn,
                         preferred_element_type=jnp.float32,
                         precision=lax.Precision.HIGHEST)
    mp = lax.dot_general(mt, w[:, HALF:], dn,
                         preferred_element_type=jnp.float32,
                         precision=lax.Precision.HIGHEST) + b
    comb = yp[:, None, :] + mp[None, :, :]
    out_ref[...] = comb.reshape(YEAR_DIM * MONTH_PAD, D_MODEL)


def _build_table(year_table, month_table_padded, w, b2d):
    return pl.pallas_call(
        _table_body,
        out_shape=jax.ShapeDtypeStruct((YEAR_DIM * MONTH_PAD, D_MODEL),
                                       jnp.float32),
    )(year_table, month_table_padded, w, b2d)


def _fuse_body(yi_ref, mi_ref, out_ref):
    ci = yi_ref[...] * MONTH_PAD + mi_ref[...]
    out_ref[...] = ci.reshape(1, L_SEQ, 128)


def _fuse_indices(yi_t, mi_t):
    return pl.pallas_call(
        _fuse_body,
        grid=(KB_TOTAL,),
        in_specs=[
            pl.BlockSpec((L_SEQ, 128), lambda k: (0, k)),
            pl.BlockSpec((L_SEQ, 128), lambda k: (0, k)),
        ],
        out_specs=pl.BlockSpec((1, L_SEQ, 128), lambda k: (k, 0, 0)),
        out_shape=jax.ShapeDtypeStruct((KB_TOTAL, L_SEQ, 128), jnp.int32),
    )(yi_t, mi_t)


_SC_MESH = plsc.VectorSubcoreMesh(core_axis_name="c", subcore_axis_name="s")


def _iota16():
    return lax.broadcasted_iota(jnp.int32, (16,), 0)


@functools.partial(
    pl.kernel,
    out_type=jax.ShapeDtypeStruct((N_ROWS, D_MODEL), jnp.float32),
    mesh=_SC_MESH,
    compiler_params=pltpu.CompilerParams(use_tc_tiling_on_sc=False,
                                         needs_layout_passes=False),
    scratch_types=[
        pltpu.VMEM((L_SEQ, 128), jnp.int32),
        pltpu.VMEM((L_SEQ, 128), jnp.int32),
        pltpu.VMEM((SUB_ROWS, D_MODEL), jnp.float32),
        pltpu.VMEM((SUB_ROWS, D_MODEL), jnp.float32),
        pltpu.SemaphoreType.DMA,
        pltpu.SemaphoreType.DMA,
        pltpu.SemaphoreType.DMA,
    ],
)
def _sc_lookup(table_hbm, ci3_hbm, out_hbm, tile_v, cir_v, rows0, rows1,
               sem_g, sem_w0, sem_w1):
    wid = lax.axis_index("s") * 2 + lax.axis_index("c")
    iota = _iota16()

    def tile_body(t, carry):
        kb = wid * KB_PER_W + t
        pltpu.sync_copy(ci3_hbm.at[kb], tile_v)

        def pair_body(p, c2):
            b0 = p * 2
            for k in range(25):
                if k < 12:
                    rows_i = iota + (16 * k)
                    cols_i = jnp.zeros((16,), jnp.int32) + b0
                elif k == 12:
                    la = iota + 192
                    wrap = la >= L_SEQ
                    rows_i = la - jnp.where(wrap, L_SEQ, 0)
                    cols_i = jnp.where(wrap, 1, 0) + b0
                else:
                    rows_i = iota + (16 * k - L_SEQ)
                    cols_i = jnp.zeros((16,), jnp.int32) + (b0 + 1)
                vals = plsc.load_gather(tile_v, [rows_i, cols_i])
                r0 = p * 400 + 16 * k
                cir_v[r0 // 128, pl.ds(lax.rem(r0, 128), 16)] = vals
            return c2

        lax.fori_loop(0, 64, pair_body, 0)

        out_base = kb * TILE_ROWS

        def pipe_body(pg, c3):
            for half, rows_v, sem_w in ((0, rows0, sem_w0), (1, rows1, sem_w1)):
                sub = pg * 2 + half
                r_base = out_base + sub * SUB_ROWS

                @pl.when(jnp.logical_or(t > 0, pg > 0))
                def _wait_prev():
                    pltpu.make_async_copy(
                        rows_v, out_hbm.at[pl.ds(r_base, SUB_ROWS)],
                        sem_w).wait()

                copies = [
                    pltpu.async_copy(
                        table_hbm.at[cir_v.at[sub * SUB_IDX + j]],
                        rows_v.at[pl.ds(j * 128, 128)],
                        sem_g,
                    )
                    for j in range(SUB_IDX)
                ]
                for c in copies:
                    c.wait()
                pltpu.async_copy(
                    rows_v, out_hbm.at[pl.ds(r_base, SUB_ROWS)], sem_w)
            return c3

        lax.fori_loop(0, PAIRS_PER_TILE, pipe_body, 0)
        return carry

    lax.fori_loop(0, KB_PER_W, tile_body, 0)
    pltpu.make_async_copy(rows0, out_hbm.at[pl.ds(0, SUB_ROWS)], sem_w0).wait()
    pltpu.make_async_copy(rows1, out_hbm.at[pl.ds(0, SUB_ROWS)], sem_w1).wait()


def kernel(year_indices, month_indices, year_table, month_table, W, b):
    mt_pad = jnp.zeros((MONTH_PAD, HALF), jnp.float32).at[:12].set(month_table)
    table = _build_table(year_table, mt_pad, W, b.reshape(1, D_MODEL))
    ci3 = _fuse_indices(year_indices.T.astype(jnp.int32),
                        month_indices.T.astype(jnp.int32))
    return _sc_lookup(table, ci3)
